# Initial kernel scaffold; baseline (speedup 1.0000x reference)
import functools

import jax
import jax.numpy as jnp
import numpy as np
from jax.experimental import pallas as pl
from jax.experimental.pallas import tpu as pltpu

HH = 64
WW = 64
NA = 15
NPOS = HH * WW
NANCH = NPOS * NA
N_PRE = 12000
N_POST = 2000
NMS_THRESH = 0.7
MIN_SIZE = 16.0
INT_MIN = jnp.int32(-(2 ** 31))
NEG_INF = jnp.float32(-jnp.inf)


def _anchor_base():
    base_size = 16.0
    ratios = [0.5, 1.0, 2.0]
    anchor_scales = [2, 4, 8, 16, 32]
    py = base_size / 2.0
    px = base_size / 2.0
    ab = np.zeros((len(ratios) * len(anchor_scales), 4), dtype=np.float32)
    for i, r in enumerate(ratios):
        for j, s in enumerate(anchor_scales):
            h = base_size * s * np.sqrt(r)
            w = base_size * s * np.sqrt(1.0 / r)
            k = i * len(anchor_scales) + j
            ab[k, 0] = py - h / 2.0
            ab[k, 1] = px - w / 2.0
            ab[k, 2] = py + h / 2.0
            ab[k, 3] = px + w / 2.0
    return ab


def _anchors_np():
    ab = _anchor_base()
    shift_y = np.arange(0, HH * 16, 16)
    shift_x = np.arange(0, WW * 16, 16)
    sx, sy = np.meshgrid(shift_x, shift_y)
    shift = np.stack((sy.ravel(), sx.ravel(), sy.ravel(), sx.ravel()), axis=1).astype(np.float32)
    anchor = shift.reshape(-1, 1, 4) + ab.reshape(1, NA, 4)
    return anchor.reshape(NANCH, 4)


_ANCHORS = _anchors_np()
_SRC_H_FLAT = _ANCHORS[:, 2] - _ANCHORS[:, 0]
_SRC_W_FLAT = _ANCHORS[:, 3] - _ANCHORS[:, 1]
_SRC_CY_FLAT = _ANCHORS[:, 0] + np.float32(0.5) * _SRC_H_FLAT
_SRC_CX_FLAT = _ANCHORS[:, 1] + np.float32(0.5) * _SRC_W_FLAT
_SRC_H = _SRC_H_FLAT.reshape(NPOS, NA).T.copy()
_SRC_W = _SRC_W_FLAT.reshape(NPOS, NA).T.copy()
_SRC_CY = _SRC_CY_FLAT.reshape(NPOS, NA).T.copy()
_SRC_CX = _SRC_CX_FLAT.reshape(NPOS, NA).T.copy()


def _head_body(x9_ref, w1_ref, b1_ref, ws_ref, bs_ref, wl_ref, bl_ref,
               s_ref, l_ref):
    h = jnp.dot(w1_ref[...], x9_ref[...], preferred_element_type=jnp.float32)
    h = jax.nn.relu(h + b1_ref[...])
    s_ref[...] = jnp.dot(ws_ref[...], h, preferred_element_type=jnp.float32) + bs_ref[...]
    l_ref[...] = jnp.dot(wl_ref[...], h, preferred_element_type=jnp.float32) + bl_ref[...]


def _run_head(x9, w1, b1, ws, bs, wl, bl):
    nb = 8
    blk = NPOS // nb
    return pl.pallas_call(
        _head_body,
        grid=(nb,),
        in_specs=[
            pl.BlockSpec((2304, blk), lambda j: (0, j)),
            pl.BlockSpec((256, 2304), lambda j: (0, 0)),
            pl.BlockSpec((256, 1), lambda j: (0, 0)),
            pl.BlockSpec((30, 256), lambda j: (0, 0)),
            pl.BlockSpec((30, 1), lambda j: (0, 0)),
            pl.BlockSpec((60, 256), lambda j: (0, 0)),
            pl.BlockSpec((60, 1), lambda j: (0, 0)),
        ],
        out_specs=[
            pl.BlockSpec((30, blk), lambda j: (0, j)),
            pl.BlockSpec((60, blk), lambda j: (0, j)),
        ],
        out_shape=[
            jax.ShapeDtypeStruct((30, NPOS), jnp.float32),
            jax.ShapeDtypeStruct((60, NPOS), jnp.float32),
        ],
    )(x9, w1, b1, ws, bs, wl, bl)


def _decode_body(hw_ref, fg_ref, dy_ref, dx_ref, dh_ref, dw_ref,
                 sh_ref, sw_ref, scy_ref, scx_ref,
                 y0_ref, x0_ref, y1_ref, x1_ref, area_ref, ss_ref):
    H = hw_ref[0]
    W = hw_ref[1]
    src_h = sh_ref[...]
    src_w = sw_ref[...]
    cy = dy_ref[...] * src_h + scy_ref[...]
    cx = dx_ref[...] * src_w + scx_ref[...]
    h = jnp.exp(dh_ref[...]) * src_h
    w = jnp.exp(dw_ref[...]) * src_w
    half = jnp.float32(0.5)
    y0 = jnp.minimum(jnp.maximum(cy - half * h, 0.0), H)
    x0 = jnp.minimum(jnp.maximum(cx - half * w, 0.0), W)
    y1 = jnp.minimum(jnp.maximum(cy + half * h, 0.0), H)
    x1 = jnp.minimum(jnp.maximum(cx + half * w, 0.0), W)
    hs = y1 - y0
    ws = x1 - x0
    s0 = jnp.where((hs >= MIN_SIZE) & (ws >= MIN_SIZE), fg_ref[...], NEG_INF)

    b = jax.lax.bitcast_convert_type(s0, jnp.int32)
    mono = jnp.where(b < 0, jnp.bitwise_xor(jnp.bitwise_not(b), INT_MIN), b)
    cnt_nonneg = jnp.sum((mono >= 0).astype(jnp.int32))
    p0 = jnp.where(cnt_nonneg >= N_PRE, jnp.int32(0), INT_MIN)

    def bit_step(i, p):
        trial = p | jax.lax.shift_left(jnp.int32(1), jnp.int32(30) - i)
        cnt = jnp.sum((mono >= trial).astype(jnp.int32))
        return jnp.where(cnt >= N_PRE, trial, p)

    t = jax.lax.fori_loop(0, 31, bit_step, p0)

    y0_ref[...] = y0
    x0_ref[...] = x0
    y1_ref[...] = y1
    x1_ref[...] = x1
    area_ref[...] = hs * ws
    ss_ref[...] = jnp.where(mono >= t, s0, NEG_INF)


def _run_decode(hw, fg, dy, dx, dh, dw):
    planar = pl.BlockSpec((NA, NPOS), lambda: (0, 0))
    consts = (jnp.asarray(_SRC_H), jnp.asarray(_SRC_W),
              jnp.asarray(_SRC_CY), jnp.asarray(_SRC_CX))
    return pl.pallas_call(
        _decode_body,
        in_specs=[pl.BlockSpec(memory_space=pltpu.SMEM)] + [planar] * 9,
        out_specs=[planar] * 6,
        out_shape=[jax.ShapeDtypeStruct((NA, NPOS), jnp.float32)] * 6,
    )(hw, fg, dy, dx, dh, dw, *consts)


def _nms_body(y0_ref, x0_ref, y1_ref, x1_ref, area_ref, ss_ref, out_ref):
    Y0 = y0_ref[...]
    X0 = x0_ref[...]
    Y1 = y1_ref[...]
    X1 = x1_ref[...]
    AREA = area_ref[...]
    lin = (jax.lax.broadcasted_iota(jnp.int32, (NA, NPOS), 0) * NPOS
           + jax.lax.broadcasted_iota(jnp.int32, (NA, NPOS), 1))
    BIG = jnp.int32(2 ** 30)
    lane4 = jax.lax.broadcasted_iota(jnp.int32, (1, 4), 1)

    def body(i, carry):
        s, b0 = carry
        m = jnp.max(s)
        idx = jnp.min(jnp.where(s == m, lin, BIG))
        b0 = jnp.where(i == 0, idx, b0)
        pick = jnp.where(m > NEG_INF, idx, b0)
        onehot = lin == pick
        y0p = jnp.sum(jnp.where(onehot, Y0, 0.0))
        x0p = jnp.sum(jnp.where(onehot, X0, 0.0))
        y1p = jnp.sum(jnp.where(onehot, Y1, 0.0))
        x1p = jnp.sum(jnp.where(onehot, X1, 0.0))
        areap = jnp.sum(jnp.where(onehot, AREA, 0.0))
        tl_y = jnp.maximum(Y0, y0p)
        tl_x = jnp.maximum(X0, x0p)
        br_y = jnp.minimum(Y1, y1p)
        br_x = jnp.minimum(X1, x1p)
        inter = (jnp.maximum(br_y - tl_y, 0.0) * jnp.maximum(br_x - tl_x, 0.0))
        iou = inter / (areap + AREA - inter + 1e-9)
        s = jnp.where(iou >= NMS_THRESH, NEG_INF, s)
        row = jnp.where(lane4 == 0, y0p,
              jnp.where(lane4 == 1, x0p,
              jnp.where(lane4 == 2, y1p, x1p)))
        out_ref[pl.ds(i, 1), :] = row.astype(jnp.float32)
        return s, b0

    jax.lax.fori_loop(0, N_POST, body, (ss_ref[...], jnp.int32(0)))


def _run_nms(y0, x0, y1, x1, area, ss):
    planar = pl.BlockSpec((NA, NPOS), lambda: (0, 0))
    return pl.pallas_call(
        _nms_body,
        in_specs=[planar] * 6,
        out_specs=pl.BlockSpec((N_POST, 4), lambda: (0, 0)),
        out_shape=jax.ShapeDtypeStruct((N_POST, 4), jnp.float32),
    )(y0, x0, y1, x1, area, ss)


@jax.jit
def kernel(x, img_size, conv1_w, conv1_b, score_w, score_b, loc_w, loc_b):
    n = x.shape[0]
    xp = jnp.pad(x[0], ((0, 0), (1, 1), (1, 1)))
    cols = [xp[:, dy:dy + HH, dx:dx + WW].reshape(256, NPOS)
            for dy in range(3) for dx in range(3)]
    x9 = jnp.concatenate(cols, axis=0)
    w1 = jnp.transpose(conv1_w, (0, 2, 3, 1)).reshape(256, 2304)
    ws = score_w.reshape(30, 256)
    wl = loc_w.reshape(60, 256)
    s_m, l_m = _run_head(x9, w1, conv1_b[:, None], ws, score_b[:, None],
                         wl, loc_b[:, None])

    rpn_scores = s_m.T.reshape(n, NANCH, 2)
    rpn_locs = l_m.T.reshape(n, NANCH, 4)

    fg = s_m[1::2]
    dy = l_m[0::4]
    dx = l_m[1::4]
    dh = l_m[2::4]
    dw = l_m[3::4]
    hw = img_size.astype(jnp.float32)

    y0, x0, y1, x1, area, ss = _run_decode(hw, fg, dy, dx, dh, dw)
    rois = _run_nms(y0, x0, y1, x1, area, ss)

    roi_indices = jnp.zeros((N_POST,), dtype=jnp.int32)
    anchor = jnp.asarray(_ANCHORS)
    return (rpn_locs, rpn_scores, rois, roi_indices, anchor)


# speedup vs baseline: 8.9223x; 8.9223x over previous
import functools

import jax
import jax.numpy as jnp
import numpy as np
from jax.experimental import pallas as pl
from jax.experimental.pallas import tpu as pltpu

HH = 64
WW = 64
NA = 15
NPOS = HH * WW
NANCH = NPOS * NA
N_PRE = 12000
N_POST = 2000
NMS_THRESH = 0.7
MIN_SIZE = 16.0
INT_MIN = -(2 ** 31)
NEG_INF = float("-inf")


def _anchor_base():
    base_size = 16.0
    ratios = [0.5, 1.0, 2.0]
    anchor_scales = [2, 4, 8, 16, 32]
    py = base_size / 2.0
    px = base_size / 2.0
    ab = np.zeros((len(ratios) * len(anchor_scales), 4), dtype=np.float32)
    for i, r in enumerate(ratios):
        for j, s in enumerate(anchor_scales):
            h = base_size * s * np.sqrt(r)
            w = base_size * s * np.sqrt(1.0 / r)
            k = i * len(anchor_scales) + j
            ab[k, 0] = py - h / 2.0
            ab[k, 1] = px - w / 2.0
            ab[k, 2] = py + h / 2.0
            ab[k, 3] = px + w / 2.0
    return ab


def _anchors_np():
    ab = _anchor_base()
    shift_y = np.arange(0, HH * 16, 16)
    shift_x = np.arange(0, WW * 16, 16)
    sx, sy = np.meshgrid(shift_x, shift_y)
    shift = np.stack((sy.ravel(), sx.ravel(), sy.ravel(), sx.ravel()), axis=1).astype(np.float32)
    anchor = shift.reshape(-1, 1, 4) + ab.reshape(1, NA, 4)
    return anchor.reshape(NANCH, 4)


_ANCHORS = _anchors_np()
_SRC_H_FLAT = _ANCHORS[:, 2] - _ANCHORS[:, 0]
_SRC_W_FLAT = _ANCHORS[:, 3] - _ANCHORS[:, 1]
_SRC_CY_FLAT = _ANCHORS[:, 0] + np.float32(0.5) * _SRC_H_FLAT
_SRC_CX_FLAT = _ANCHORS[:, 1] + np.float32(0.5) * _SRC_W_FLAT
_SRC_H = _SRC_H_FLAT.reshape(NPOS, NA).T.copy()
_SRC_W = _SRC_W_FLAT.reshape(NPOS, NA).T.copy()
_SRC_CY = _SRC_CY_FLAT.reshape(NPOS, NA).T.copy()
_SRC_CX = _SRC_CX_FLAT.reshape(NPOS, NA).T.copy()


def _head_body(x9_ref, w1_ref, b1_ref, ws_ref, bs_ref, wl_ref, bl_ref,
               s_ref, l_ref):
    h = jnp.dot(w1_ref[...], x9_ref[...], preferred_element_type=jnp.float32)
    h = jax.nn.relu(h + b1_ref[...])
    s_ref[...] = jnp.dot(ws_ref[...], h, preferred_element_type=jnp.float32) + bs_ref[...]
    l_ref[...] = jnp.dot(wl_ref[...], h, preferred_element_type=jnp.float32) + bl_ref[...]


def _run_head(x9, w1, b1, ws, bs, wl, bl):
    nb = 8
    blk = NPOS // nb
    return pl.pallas_call(
        _head_body,
        grid=(nb,),
        in_specs=[
            pl.BlockSpec((2304, blk), lambda j: (0, j)),
            pl.BlockSpec((256, 2304), lambda j: (0, 0)),
            pl.BlockSpec((256, 1), lambda j: (0, 0)),
            pl.BlockSpec((30, 256), lambda j: (0, 0)),
            pl.BlockSpec((30, 1), lambda j: (0, 0)),
            pl.BlockSpec((60, 256), lambda j: (0, 0)),
            pl.BlockSpec((60, 1), lambda j: (0, 0)),
        ],
        out_specs=[
            pl.BlockSpec((30, blk), lambda j: (0, j)),
            pl.BlockSpec((60, blk), lambda j: (0, j)),
        ],
        out_shape=[
            jax.ShapeDtypeStruct((30, NPOS), jnp.float32),
            jax.ShapeDtypeStruct((60, NPOS), jnp.float32),
        ],
    )(x9, w1, b1, ws, bs, wl, bl)


def _decode_body(hw_ref, fg_ref, dy_ref, dx_ref, dh_ref, dw_ref,
                 sh_ref, sw_ref, scy_ref, scx_ref,
                 y0_ref, x0_ref, y1_ref, x1_ref, area_ref, ss_ref):
    H = hw_ref[0]
    W = hw_ref[1]
    src_h = sh_ref[...]
    src_w = sw_ref[...]
    cy = dy_ref[...] * src_h + scy_ref[...]
    cx = dx_ref[...] * src_w + scx_ref[...]
    h = jnp.exp(dh_ref[...]) * src_h
    w = jnp.exp(dw_ref[...]) * src_w
    half = jnp.float32(0.5)
    y0 = jnp.minimum(jnp.maximum(cy - half * h, 0.0), H)
    x0 = jnp.minimum(jnp.maximum(cx - half * w, 0.0), W)
    y1 = jnp.minimum(jnp.maximum(cy + half * h, 0.0), H)
    x1 = jnp.minimum(jnp.maximum(cx + half * w, 0.0), W)
    hs = y1 - y0
    ws = x1 - x0
    s0 = jnp.where((hs >= MIN_SIZE) & (ws >= MIN_SIZE), fg_ref[...], NEG_INF)

    b = jax.lax.bitcast_convert_type(s0, jnp.int32)
    mono = jnp.where(b < 0, jnp.bitwise_xor(jnp.bitwise_not(b), jnp.int32(INT_MIN)), b)
    cnt_nonneg = jnp.sum((mono >= 0).astype(jnp.int32))
    p0 = jnp.where(cnt_nonneg >= N_PRE, jnp.int32(0), jnp.int32(INT_MIN))

    def bit_step(i, p):
        trial = p | jax.lax.shift_left(jnp.int32(1), jnp.int32(30) - i)
        cnt = jnp.sum((mono >= trial).astype(jnp.int32))
        return jnp.where(cnt >= N_PRE, trial, p)

    t = jax.lax.fori_loop(0, 31, bit_step, p0)

    y0_ref[...] = y0
    x0_ref[...] = x0
    y1_ref[...] = y1
    x1_ref[...] = x1
    area_ref[...] = hs * ws
    ss_ref[...] = jnp.where(mono >= t, s0, NEG_INF)


def _run_decode(hw, fg, dy, dx, dh, dw):
    planar = pl.BlockSpec((NA, NPOS), lambda: (0, 0))
    consts = (jnp.asarray(_SRC_H), jnp.asarray(_SRC_W),
              jnp.asarray(_SRC_CY), jnp.asarray(_SRC_CX))
    return pl.pallas_call(
        _decode_body,
        in_specs=[pl.BlockSpec(memory_space=pltpu.SMEM)] + [planar] * 9,
        out_specs=[planar] * 6,
        out_shape=[jax.ShapeDtypeStruct((NA, NPOS), jnp.float32)] * 6,
    )(hw, fg, dy, dx, dh, dw, *consts)


def _nms_body(y0_ref, x0_ref, y1_ref, x1_ref, area_ref, ss_ref, out_ref):
    Y0 = y0_ref[...]
    X0 = x0_ref[...]
    Y1 = y1_ref[...]
    X1 = x1_ref[...]
    AREA = area_ref[...]
    lin = (jax.lax.broadcasted_iota(jnp.int32, (NA, NPOS), 0) * NPOS
           + jax.lax.broadcasted_iota(jnp.int32, (NA, NPOS), 1))
    BIG = jnp.int32(2 ** 30)
    lane4 = jax.lax.broadcasted_iota(jnp.int32, (1, 4), 1)

    def body(i, carry):
        s, b0 = carry
        m = jnp.max(s)
        idx = jnp.min(jnp.where(s == m, lin, BIG))
        b0 = jnp.where(i == 0, idx, b0)
        pick = jnp.where(m > NEG_INF, idx, b0)
        onehot = lin == pick
        y0p = jnp.sum(jnp.where(onehot, Y0, 0.0))
        x0p = jnp.sum(jnp.where(onehot, X0, 0.0))
        y1p = jnp.sum(jnp.where(onehot, Y1, 0.0))
        x1p = jnp.sum(jnp.where(onehot, X1, 0.0))
        areap = jnp.sum(jnp.where(onehot, AREA, 0.0))
        tl_y = jnp.maximum(Y0, y0p)
        tl_x = jnp.maximum(X0, x0p)
        br_y = jnp.minimum(Y1, y1p)
        br_x = jnp.minimum(X1, x1p)
        inter = (jnp.maximum(br_y - tl_y, 0.0) * jnp.maximum(br_x - tl_x, 0.0))
        iou = inter / (areap + AREA - inter + 1e-9)
        s = jnp.where(iou >= NMS_THRESH, NEG_INF, s)
        row = jnp.where(lane4 == 0, y0p,
              jnp.where(lane4 == 1, x0p,
              jnp.where(lane4 == 2, y1p, x1p)))
        out_ref[pl.ds(i, 1), :] = row.astype(jnp.float32)
        return s, b0

    jax.lax.fori_loop(0, N_POST, body, (ss_ref[...], jnp.int32(0)))


def _run_nms(y0, x0, y1, x1, area, ss):
    planar = pl.BlockSpec((NA, NPOS), lambda: (0, 0))
    return pl.pallas_call(
        _nms_body,
        in_specs=[planar] * 6,
        out_specs=pl.BlockSpec((N_POST, 4), lambda: (0, 0)),
        out_shape=jax.ShapeDtypeStruct((N_POST, 4), jnp.float32),
    )(y0, x0, y1, x1, area, ss)


@jax.jit
def kernel(x, img_size, conv1_w, conv1_b, score_w, score_b, loc_w, loc_b):
    n = x.shape[0]
    xp = jnp.pad(x[0], ((0, 0), (1, 1), (1, 1)))
    cols = [xp[:, dy:dy + HH, dx:dx + WW].reshape(256, NPOS)
            for dy in range(3) for dx in range(3)]
    x9 = jnp.concatenate(cols, axis=0)
    w1 = jnp.transpose(conv1_w, (0, 2, 3, 1)).reshape(256, 2304)
    ws = score_w.reshape(30, 256)
    wl = loc_w.reshape(60, 256)
    s_m, l_m = _run_head(x9, w1, conv1_b[:, None], ws, score_b[:, None],
                         wl, loc_b[:, None])

    rpn_scores = s_m.T.reshape(n, NANCH, 2)
    rpn_locs = l_m.T.reshape(n, NANCH, 4)

    fg = s_m[1::2]
    dy = l_m[0::4]
    dx = l_m[1::4]
    dh = l_m[2::4]
    dw = l_m[3::4]
    hw = img_size.astype(jnp.float32)

    y0, x0, y1, x1, area, ss = _run_decode(hw, fg, dy, dx, dh, dw)
    rois = _run_nms(y0, x0, y1, x1, area, ss)

    roi_indices = jnp.zeros((N_POST,), dtype=jnp.int32)
    anchor = jnp.asarray(_ANCHORS)
    return (rpn_locs, rpn_scores, rois, roi_indices, anchor)
